# baseline (speedup 1.0000x reference)
import functools

import jax
import jax.numpy as jnp
from jax.experimental import pallas as pl
from jax.experimental.pallas import tpu as pltpu

B, L, E = 1, 2048, 768
N_EXPERTS = 8
TOP_K = 2
MLP_DIM = 2048

CAP = L
ROW_BLK = 512
BLKS_PER_E = CAP // ROW_BLK
N_SLOTS = N_EXPERTS * CAP
CHUNK = 1024


def _routing_kernel(x_ref, gate_ref, src_ref, prob_ref, cnt_ref):
    logitsT = jax.lax.dot_general(
        gate_ref[...], x_ref[...], (((0,), (1,)), ((), ())),
        preferred_element_type=jnp.float32)
    sub = jax.lax.broadcasted_iota(jnp.int32, (N_EXPERTS, L), 0)
    a1 = jnp.argmax(logitsT, axis=0)[None, :]
    m1 = jnp.max(logitsT, axis=0, keepdims=True)
    masked = jnp.where(sub == a1, -jnp.inf, logitsT)
    a2 = jnp.argmax(masked, axis=0)[None, :]
    m2 = jnp.max(masked, axis=0, keepdims=True)
    e2 = jnp.exp(m2 - m1)
    p1 = 1.0 / (1.0 + e2)
    p2 = e2 / (1.0 + e2)

    a_mat = ((sub == a1) | (sub == a2)).astype(jnp.bfloat16)
    l_row = jax.lax.broadcasted_iota(jnp.int32, (L, L), 0)
    l_col = jax.lax.broadcasted_iota(jnp.int32, (L, L), 1)
    upper = (l_row < l_col).astype(jnp.bfloat16)
    rankT = jax.lax.dot_general(
        a_mat, upper, (((1,), (0,)), ((), ())),
        preferred_element_type=jnp.float32)
    cnt = rankT[:, L - 1:L] + a_mat[:, L - 1:L].astype(jnp.float32)
    cnt_ref[...] = cnt.astype(jnp.int32)

    nsel = jax.lax.broadcasted_iota(jnp.int32, (N_EXPERTS, L), 0)
    r1 = jnp.sum(jnp.where(nsel == a1, rankT, 0.0), axis=0, keepdims=True)
    r2 = jnp.sum(jnp.where(nsel == a2, rankT, 0.0), axis=0, keepdims=True)
    d1 = a1 * CAP + r1.astype(jnp.int32)
    d2 = a2 * CAP + r2.astype(jnp.int32)

    lane_tok = jax.lax.broadcasted_iota(
        jnp.int32, (CHUNK, L), 1).astype(jnp.float32)
    slot0 = jax.lax.broadcasted_iota(jnp.int32, (CHUNK, L), 0)
    for c in range(N_SLOTS // CHUNK):
        slot = slot0 + c * CHUNK
        hit1 = (slot == d1).astype(jnp.float32)
        hit2 = (slot == d2).astype(jnp.float32)
        hits = hit1 + hit2
        src_c = jnp.sum(hits * lane_tok, axis=1, keepdims=True)
        prob_c = jnp.sum(hit1 * p1 + hit2 * p2, axis=1, keepdims=True)
        src_ref[c * CHUNK:(c + 1) * CHUNK, :] = src_c
        prob_ref[c * CHUNK:(c + 1) * CHUNK, :] = prob_c


def _group_mlp_kernel(cnt_ref, src_ref, prob_ref, x_ref,
                      w0_ref, w1_ref, wo_ref, out_ref):
    n = pl.program_id(0)
    i = pl.program_id(1)

    @pl.when((n == 0) & (i == 0))
    def _init():
        out_ref[...] = jnp.zeros_like(out_ref)

    @pl.when(i * ROW_BLK < cnt_ref[n])
    def _active():
        src = src_ref[0, 0, :]
        tok = jax.lax.broadcasted_iota(jnp.int32, (L, ROW_BLK), 0)
        sel = tok == src[None, :]
        gt_b = sel.astype(jnp.bfloat16)
        x_b = x_ref[...].astype(jnp.bfloat16)
        xs = jax.lax.dot_general(
            gt_b, x_b, (((0,), (0,)), ((), ())),
            preferred_element_type=jnp.float32).astype(jnp.bfloat16)
        h0 = jnp.dot(xs, w0_ref[0].astype(jnp.bfloat16),
                     preferred_element_type=jnp.float32)
        h1 = jnp.dot(xs, w1_ref[0].astype(jnp.bfloat16),
                     preferred_element_type=jnp.float32)
        m = ((h0 * jax.nn.sigmoid(h0)) * h1).astype(jnp.bfloat16)
        y = jnp.dot(m, wo_ref[0].astype(jnp.bfloat16),
                    preferred_element_type=jnp.float32)
        gt_w = sel.astype(jnp.float32) * prob_ref[0, 0, :][None, :]
        out_ref[...] += jnp.dot(gt_w, y, preferred_element_type=jnp.float32)


@jax.jit
def _moe(inputs, gate_kernel, w0_kernel, w1_kernel, wo_kernel):
    x = inputs.reshape(L, E).astype(jnp.float32)

    src_f, prob_f, cnt = pl.pallas_call(
        _routing_kernel,
        grid=(1,),
        in_specs=[
            pl.BlockSpec((L, E), lambda g: (0, 0)),
            pl.BlockSpec((E, N_EXPERTS), lambda g: (0, 0)),
        ],
        out_specs=[
            pl.BlockSpec((N_SLOTS, 1), lambda g: (0, 0)),
            pl.BlockSpec((N_SLOTS, 1), lambda g: (0, 0)),
            pl.BlockSpec((N_EXPERTS, 1), lambda g: (0, 0)),
        ],
        out_shape=[
            jax.ShapeDtypeStruct((N_SLOTS, 1), jnp.float32),
            jax.ShapeDtypeStruct((N_SLOTS, 1), jnp.float32),
            jax.ShapeDtypeStruct((N_EXPERTS, 1), jnp.int32),
        ],
    )(x, gate_kernel)

    src = src_f.astype(jnp.int32).reshape(N_SLOTS // ROW_BLK, 1, ROW_BLK)
    probs = prob_f.reshape(N_SLOTS // ROW_BLK, 1, ROW_BLK)
    cnt = cnt.reshape(N_EXPERTS)

    out = pl.pallas_call(
        _group_mlp_kernel,
        grid_spec=pltpu.PrefetchScalarGridSpec(
            num_scalar_prefetch=1,
            grid=(N_EXPERTS, BLKS_PER_E),
            in_specs=[
                pl.BlockSpec((1, 1, ROW_BLK),
                             lambda n, i, s: (n * BLKS_PER_E + i, 0, 0)),
                pl.BlockSpec((1, 1, ROW_BLK),
                             lambda n, i, s: (n * BLKS_PER_E + i, 0, 0)),
                pl.BlockSpec((L, E), lambda n, i, s: (0, 0)),
                pl.BlockSpec((1, E, MLP_DIM), lambda n, i, s: (n, 0, 0)),
                pl.BlockSpec((1, E, MLP_DIM), lambda n, i, s: (n, 0, 0)),
                pl.BlockSpec((1, MLP_DIM, E), lambda n, i, s: (n, 0, 0)),
            ],
            out_specs=pl.BlockSpec((L, E), lambda n, i, s: (0, 0)),
        ),
        out_shape=jax.ShapeDtypeStruct((L, E), jnp.float32),
        compiler_params=pltpu.CompilerParams(
            dimension_semantics=("arbitrary", "arbitrary"),
        ),
    )(cnt, src, probs, x, w0_kernel, w1_kernel, wo_kernel)
    return out.reshape(B, L, E)


def kernel(inputs, gate_kernel, w0_kernel, w1_kernel, wo_kernel):
    return _moe(inputs, gate_kernel, w0_kernel, w1_kernel, wo_kernel)


# speedup vs baseline: 1.3234x; 1.3234x over previous
import jax
import jax.numpy as jnp
from jax.experimental import pallas as pl
from jax.experimental.pallas import tpu as pltpu

B, L, E = 1, 2048, 768
N_EXPERTS = 8
TOP_K = 2
MLP_DIM = 2048

CAP = L
ROW_BLK = 512
BLKS_PER_E = CAP // ROW_BLK


def _routing_kernel(x_ref, gate_ref, d1_ref, d2_ref, p1_ref, p2_ref, cnt_ref):
    logits = jnp.dot(x_ref[...], gate_ref[...],
                     preferred_element_type=jnp.float32)
    lane = jax.lax.broadcasted_iota(jnp.int32, (L, N_EXPERTS), 1)
    a1 = jnp.argmax(logits, axis=-1)[:, None]
    m1 = jnp.max(logits, axis=-1, keepdims=True)
    masked = jnp.where(lane == a1, -jnp.inf, logits)
    a2 = jnp.argmax(masked, axis=-1)[:, None]
    m2 = jnp.max(masked, axis=-1, keepdims=True)
    e2 = jnp.exp(m2 - m1)
    p1_ref[...] = 1.0 / (1.0 + e2)
    p2_ref[...] = e2 / (1.0 + e2)

    a_mat = ((lane == a1) | (lane == a2)).astype(jnp.bfloat16)
    l_row = jax.lax.broadcasted_iota(jnp.int32, (L, L), 0)
    l_col = jax.lax.broadcasted_iota(jnp.int32, (L, L), 1)
    lower = (l_row > l_col).astype(jnp.bfloat16)
    rank = jnp.dot(lower, a_mat,
                   preferred_element_type=jnp.float32)
    cnt = rank[L - 1:L, :] + a_mat[L - 1:L, :].astype(jnp.float32)
    cnt_ref[...] = cnt.astype(jnp.int32)

    r1 = jnp.sum(jnp.where(lane == a1, rank, 0.0), axis=-1, keepdims=True)
    r2 = jnp.sum(jnp.where(lane == a2, rank, 0.0), axis=-1, keepdims=True)
    d1_ref[...] = a1 * CAP + r1.astype(jnp.int32)
    d2_ref[...] = a2 * CAP + r2.astype(jnp.int32)


def _group_mlp_kernel(cnt_ref, d1_ref, d2_ref, p1_ref, p2_ref, x_ref,
                      w0_ref, w1_ref, wo_ref, out_ref):
    n = pl.program_id(0)
    i = pl.program_id(1)

    @pl.when((n == 0) & (i == 0))
    def _init():
        out_ref[...] = jnp.zeros_like(out_ref)

    @pl.when(i * ROW_BLK < cnt_ref[n])
    def _active():
        base = n * CAP + i * ROW_BLK
        slot = (jax.lax.broadcasted_iota(jnp.int32, (L, ROW_BLK), 1)
                + base)
        sel1 = slot == d1_ref[...]
        sel2 = slot == d2_ref[...]
        gt_b = (sel1 | sel2).astype(jnp.bfloat16)
        x_b = x_ref[...].astype(jnp.bfloat16)
        xs = jax.lax.dot_general(
            gt_b, x_b, (((0,), (0,)), ((), ())),
            preferred_element_type=jnp.float32).astype(jnp.bfloat16)
        h0 = jnp.dot(xs, w0_ref[0].astype(jnp.bfloat16),
                     preferred_element_type=jnp.float32)
        h1 = jnp.dot(xs, w1_ref[0].astype(jnp.bfloat16),
                     preferred_element_type=jnp.float32)
        m = ((h0 * jax.nn.sigmoid(h0)) * h1).astype(jnp.bfloat16)
        y = jnp.dot(m, wo_ref[0].astype(jnp.bfloat16),
                    preferred_element_type=jnp.float32)
        gt_w = (jnp.where(sel1, p1_ref[...], 0.0)
                + jnp.where(sel2, p2_ref[...], 0.0)).astype(jnp.bfloat16)
        out_ref[...] += jnp.dot(gt_w, y.astype(jnp.bfloat16),
                                preferred_element_type=jnp.float32)


@jax.jit
def _moe(inputs, gate_kernel, w0_kernel, w1_kernel, wo_kernel):
    x = inputs.reshape(L, E).astype(jnp.float32)

    d1, d2, p1, p2, cnt = pl.pallas_call(
        _routing_kernel,
        grid=(1,),
        in_specs=[
            pl.BlockSpec((L, E), lambda g: (0, 0)),
            pl.BlockSpec((E, N_EXPERTS), lambda g: (0, 0)),
        ],
        out_specs=[
            pl.BlockSpec((L, 1), lambda g: (0, 0)),
            pl.BlockSpec((L, 1), lambda g: (0, 0)),
            pl.BlockSpec((L, 1), lambda g: (0, 0)),
            pl.BlockSpec((L, 1), lambda g: (0, 0)),
            pl.BlockSpec((1, N_EXPERTS), lambda g: (0, 0)),
        ],
        out_shape=[
            jax.ShapeDtypeStruct((L, 1), jnp.int32),
            jax.ShapeDtypeStruct((L, 1), jnp.int32),
            jax.ShapeDtypeStruct((L, 1), jnp.float32),
            jax.ShapeDtypeStruct((L, 1), jnp.float32),
            jax.ShapeDtypeStruct((1, N_EXPERTS), jnp.int32),
        ],
    )(x, gate_kernel)

    out = pl.pallas_call(
        _group_mlp_kernel,
        grid_spec=pltpu.PrefetchScalarGridSpec(
            num_scalar_prefetch=1,
            grid=(N_EXPERTS, BLKS_PER_E),
            in_specs=[
                pl.BlockSpec((L, 1), lambda n, i, s: (0, 0)),
                pl.BlockSpec((L, 1), lambda n, i, s: (0, 0)),
                pl.BlockSpec((L, 1), lambda n, i, s: (0, 0)),
                pl.BlockSpec((L, 1), lambda n, i, s: (0, 0)),
                pl.BlockSpec((L, E), lambda n, i, s: (0, 0)),
                pl.BlockSpec((1, E, MLP_DIM), lambda n, i, s: (n, 0, 0)),
                pl.BlockSpec((1, E, MLP_DIM), lambda n, i, s: (n, 0, 0)),
                pl.BlockSpec((1, MLP_DIM, E), lambda n, i, s: (n, 0, 0)),
            ],
            out_specs=pl.BlockSpec((L, E), lambda n, i, s: (0, 0)),
        ),
        out_shape=jax.ShapeDtypeStruct((L, E), jnp.float32),
        compiler_params=pltpu.CompilerParams(
            dimension_semantics=("arbitrary", "arbitrary"),
            vmem_limit_bytes=100 * 1024 * 1024,
        ),
    )(cnt.reshape(N_EXPERTS), d1, d2, p1, p2, x,
      w0_kernel, w1_kernel, wo_kernel)
    return out.reshape(B, L, E)


def kernel(inputs, gate_kernel, w0_kernel, w1_kernel, wo_kernel):
    return _moe(inputs, gate_kernel, w0_kernel, w1_kernel, wo_kernel)


# speedup vs baseline: 1.7009x; 1.2852x over previous
import jax
import jax.numpy as jnp
from jax.experimental import pallas as pl
from jax.experimental.pallas import tpu as pltpu

B, L, E = 1, 2048, 768
N_EXPERTS = 8
TOP_K = 2
MLP_DIM = 2048

CAP = L
ROW_BLK = 512
BLKS_PER_E = CAP // ROW_BLK


def _routing_kernel(x_ref, gate_ref, d1_ref, d2_ref, p1_ref, p2_ref, cnt_ref):
    logits = jnp.dot(x_ref[...], gate_ref[...],
                     preferred_element_type=jnp.float32)
    lane = jax.lax.broadcasted_iota(jnp.int32, (L, N_EXPERTS), 1)
    a1 = jnp.argmax(logits, axis=-1)[:, None]
    m1 = jnp.max(logits, axis=-1, keepdims=True)
    masked = jnp.where(lane == a1, -jnp.inf, logits)
    a2 = jnp.argmax(masked, axis=-1)[:, None]
    m2 = jnp.max(masked, axis=-1, keepdims=True)
    e2 = jnp.exp(m2 - m1)
    p1_ref[...] = 1.0 / (1.0 + e2)
    p2_ref[...] = e2 / (1.0 + e2)

    a_mat = ((lane == a1) | (lane == a2)).astype(jnp.bfloat16)
    l_row = jax.lax.broadcasted_iota(jnp.int32, (L, L), 0)
    l_col = jax.lax.broadcasted_iota(jnp.int32, (L, L), 1)
    lower = (l_row > l_col).astype(jnp.bfloat16)
    rank = jnp.dot(lower, a_mat,
                   preferred_element_type=jnp.float32)
    cnt = rank[L - 1:L, :] + a_mat[L - 1:L, :].astype(jnp.float32)
    cnt_ref[...] = cnt.astype(jnp.int32)

    r1 = jnp.sum(jnp.where(lane == a1, rank, 0.0), axis=-1, keepdims=True)
    r2 = jnp.sum(jnp.where(lane == a2, rank, 0.0), axis=-1, keepdims=True)
    d1_ref[...] = a1 * CAP + r1.astype(jnp.int32)
    d2_ref[...] = a2 * CAP + r2.astype(jnp.int32)


def _group_mlp_kernel(cnt_ref, d1_ref, d2_ref, p1_ref, p2_ref, x_ref,
                      w0_ref, w1_ref, wo_ref, out_ref):
    n = pl.program_id(0)

    @pl.when(n == 0)
    def _init():
        out_ref[...] = jnp.zeros_like(out_ref)

    cnt = cnt_ref[n]
    iota = jax.lax.broadcasted_iota(jnp.int32, (L, ROW_BLK), 1)

    def _block(j, carry):
        @pl.when(j * ROW_BLK < cnt)
        def _active():
            slot = iota + (n * CAP + j * ROW_BLK)
            sel1 = slot == d1_ref[...]
            sel2 = slot == d2_ref[...]
            gt_b = (sel1 | sel2).astype(jnp.bfloat16)
            xs = jax.lax.dot_general(
                gt_b, x_ref[...], (((0,), (0,)), ((), ())),
                preferred_element_type=jnp.float32).astype(jnp.bfloat16)
            h0 = jnp.dot(xs, w0_ref[0].astype(jnp.bfloat16),
                         preferred_element_type=jnp.float32)
            h1 = jnp.dot(xs, w1_ref[0].astype(jnp.bfloat16),
                         preferred_element_type=jnp.float32)
            m = ((h0 * jax.nn.sigmoid(h0)) * h1).astype(jnp.bfloat16)
            y = jnp.dot(m, wo_ref[0].astype(jnp.bfloat16),
                        preferred_element_type=jnp.float32)
            gt_w = (jnp.where(sel1, p1_ref[...], 0.0)
                    + jnp.where(sel2, p2_ref[...], 0.0)).astype(jnp.bfloat16)
            out_ref[...] += jnp.dot(gt_w, y.astype(jnp.bfloat16),
                                    preferred_element_type=jnp.float32)
        return carry

    jax.lax.fori_loop(0, BLKS_PER_E, _block, 0)


@jax.jit
def _moe(inputs, gate_kernel, w0_kernel, w1_kernel, wo_kernel):
    x = inputs.reshape(L, E).astype(jnp.float32)

    d1, d2, p1, p2, cnt = pl.pallas_call(
        _routing_kernel,
        grid=(1,),
        in_specs=[
            pl.BlockSpec((L, E), lambda g: (0, 0)),
            pl.BlockSpec((E, N_EXPERTS), lambda g: (0, 0)),
        ],
        out_specs=[
            pl.BlockSpec((L, 1), lambda g: (0, 0)),
            pl.BlockSpec((L, 1), lambda g: (0, 0)),
            pl.BlockSpec((L, 1), lambda g: (0, 0)),
            pl.BlockSpec((L, 1), lambda g: (0, 0)),
            pl.BlockSpec((1, N_EXPERTS), lambda g: (0, 0)),
        ],
        out_shape=[
            jax.ShapeDtypeStruct((L, 1), jnp.int32),
            jax.ShapeDtypeStruct((L, 1), jnp.int32),
            jax.ShapeDtypeStruct((L, 1), jnp.float32),
            jax.ShapeDtypeStruct((L, 1), jnp.float32),
            jax.ShapeDtypeStruct((1, N_EXPERTS), jnp.int32),
        ],
    )(x, gate_kernel)

    out = pl.pallas_call(
        _group_mlp_kernel,
        grid_spec=pltpu.PrefetchScalarGridSpec(
            num_scalar_prefetch=1,
            grid=(N_EXPERTS,),
            in_specs=[
                pl.BlockSpec((L, 1), lambda n, s: (0, 0)),
                pl.BlockSpec((L, 1), lambda n, s: (0, 0)),
                pl.BlockSpec((L, 1), lambda n, s: (0, 0)),
                pl.BlockSpec((L, 1), lambda n, s: (0, 0)),
                pl.BlockSpec((L, E), lambda n, s: (0, 0)),
                pl.BlockSpec((1, E, MLP_DIM), lambda n, s: (n, 0, 0)),
                pl.BlockSpec((1, E, MLP_DIM), lambda n, s: (n, 0, 0)),
                pl.BlockSpec((1, MLP_DIM, E), lambda n, s: (n, 0, 0)),
            ],
            out_specs=pl.BlockSpec((L, E), lambda n, s: (0, 0)),
        ),
        out_shape=jax.ShapeDtypeStruct((L, E), jnp.float32),
        compiler_params=pltpu.CompilerParams(
            dimension_semantics=("arbitrary",),
            vmem_limit_bytes=100 * 1024 * 1024,
        ),
    )(cnt.reshape(N_EXPERTS), d1, d2, p1, p2, x.astype(jnp.bfloat16),
      w0_kernel, w1_kernel, wo_kernel)
    return out.reshape(B, L, E)


def kernel(inputs, gate_kernel, w0_kernel, w1_kernel, wo_kernel):
    return _moe(inputs, gate_kernel, w0_kernel, w1_kernel, wo_kernel)


# speedup vs baseline: 1.8538x; 1.0899x over previous
import jax
import jax.numpy as jnp
from jax.experimental import pallas as pl
from jax.experimental.pallas import tpu as pltpu

B, L, E = 1, 2048, 768
N_EXPERTS = 8
TOP_K = 2
MLP_DIM = 2048

CAP = L
ROW_BLK = 256
BLKS_PER_E = CAP // ROW_BLK


def _moe_kernel(x_ref, gate_ref, w0_ref, w1_ref, wo_ref, out_ref,
                d1_ref, d2_ref, p1_ref, p2_ref, cnt_ref):
    n = pl.program_id(0)

    @pl.when(n == 0)
    def _route():
        logits = jnp.dot(x_ref[...], gate_ref[...],
                         preferred_element_type=jnp.float32)
        lane = jax.lax.broadcasted_iota(jnp.int32, (L, N_EXPERTS), 1)
        a1 = jnp.argmax(logits, axis=-1)[:, None]
        m1 = jnp.max(logits, axis=-1, keepdims=True)
        masked = jnp.where(lane == a1, -jnp.inf, logits)
        a2 = jnp.argmax(masked, axis=-1)[:, None]
        m2 = jnp.max(masked, axis=-1, keepdims=True)
        e2 = jnp.exp(m2 - m1)
        p1_ref[...] = 1.0 / (1.0 + e2)
        p2_ref[...] = e2 / (1.0 + e2)

        a_mat = ((lane == a1) | (lane == a2)).astype(jnp.float32)
        a_b = a_mat.astype(jnp.bfloat16)
        chunks = []
        for c in range(4):
            row = (jax.lax.broadcasted_iota(jnp.int32, (L // 4, L), 0)
                   + c * (L // 4))
            col = jax.lax.broadcasted_iota(jnp.int32, (L // 4, L), 1)
            lower_c = (row > col).astype(jnp.bfloat16)
            chunks.append(jnp.dot(lower_c, a_b,
                                  preferred_element_type=jnp.float32))
        rank = jnp.concatenate(chunks, axis=0)
        csum = rank + a_mat
        r1 = jnp.sum(jnp.where(lane == a1, rank, 0.0), axis=-1,
                     keepdims=True)
        r2 = jnp.sum(jnp.where(lane == a2, rank, 0.0), axis=-1,
                     keepdims=True)
        d1_ref[...] = a1 * CAP + r1.astype(jnp.int32)
        d2_ref[...] = a2 * CAP + r2.astype(jnp.int32)
        cnt_row = csum[L - 1:L, :]
        for k in range(N_EXPERTS):
            cnt_ref[k] = cnt_row[0, k].astype(jnp.int32)
        out_ref[...] = jnp.zeros_like(out_ref)

    cnt = cnt_ref[n]
    iota = jax.lax.broadcasted_iota(jnp.int32, (L, ROW_BLK), 1)

    def _block(j, carry):
        @pl.when(j * ROW_BLK < cnt)
        def _active():
            slot = iota + (n * CAP + j * ROW_BLK)
            sel1 = slot == d1_ref[...]
            sel2 = slot == d2_ref[...]
            gt_b = (sel1 | sel2).astype(jnp.bfloat16)
            xs = jax.lax.dot_general(
                gt_b, x_ref[...].astype(jnp.bfloat16),
                (((0,), (0,)), ((), ())),
                preferred_element_type=jnp.float32).astype(jnp.bfloat16)
            h0 = jnp.dot(xs, w0_ref[0].astype(jnp.bfloat16),
                         preferred_element_type=jnp.float32)
            h1 = jnp.dot(xs, w1_ref[0].astype(jnp.bfloat16),
                         preferred_element_type=jnp.float32)
            m = ((h0 * jax.nn.sigmoid(h0)) * h1).astype(jnp.bfloat16)
            y = jnp.dot(m, wo_ref[0].astype(jnp.bfloat16),
                        preferred_element_type=jnp.float32)
            gt_w = (jnp.where(sel1, p1_ref[...], 0.0)
                    + jnp.where(sel2, p2_ref[...], 0.0)).astype(jnp.bfloat16)
            out_ref[...] += jnp.dot(gt_w, y.astype(jnp.bfloat16),
                                    preferred_element_type=jnp.float32)
        return carry

    jax.lax.fori_loop(0, BLKS_PER_E, _block, 0)


@jax.jit
def _moe(inputs, gate_kernel, w0_kernel, w1_kernel, wo_kernel):
    x = inputs.reshape(L, E).astype(jnp.float32)
    out = pl.pallas_call(
        _moe_kernel,
        grid=(N_EXPERTS,),
        in_specs=[
            pl.BlockSpec((L, E), lambda n: (0, 0)),
            pl.BlockSpec((E, N_EXPERTS), lambda n: (0, 0)),
            pl.BlockSpec((1, E, MLP_DIM), lambda n: (n, 0, 0)),
            pl.BlockSpec((1, E, MLP_DIM), lambda n: (n, 0, 0)),
            pl.BlockSpec((1, MLP_DIM, E), lambda n: (n, 0, 0)),
        ],
        out_specs=pl.BlockSpec((L, E), lambda n: (0, 0)),
        out_shape=jax.ShapeDtypeStruct((L, E), jnp.float32),
        scratch_shapes=[
            pltpu.VMEM((L, 1), jnp.int32),
            pltpu.VMEM((L, 1), jnp.int32),
            pltpu.VMEM((L, 1), jnp.float32),
            pltpu.VMEM((L, 1), jnp.float32),
            pltpu.SMEM((N_EXPERTS,), jnp.int32),
        ],
        compiler_params=pltpu.CompilerParams(
            dimension_semantics=("arbitrary",),
            vmem_limit_bytes=100 * 1024 * 1024,
        ),
    )(x, gate_kernel, w0_kernel, w1_kernel, wo_kernel)
    return out.reshape(B, L, E)


def kernel(inputs, gate_kernel, w0_kernel, w1_kernel, wo_kernel):
    return _moe(inputs, gate_kernel, w0_kernel, w1_kernel, wo_kernel)
